# baseline (speedup 1.0000x reference)
import functools

import jax
import jax.numpy as jnp
from jax import lax
from jax.experimental import pallas as pl
from jax.experimental.pallas import tpu as pltpu
from jax.experimental.pallas import tpu_sc as plsc

MOMENTUM = 0.99
NUM_CLASSES = 1000
B, D = 16384, 128
CP = 1024
NC, NS = 2, 16
NW = NC * NS
RPW = B // NW
CH = 64
NCHS = RPW // CH
CHL = 128
NCHL = RPW // CHL
TROWS = CP // NS
NQ = D // 16
NACC = 4

_mesh = plsc.VectorSubcoreMesh(
    core_axis_name="c", subcore_axis_name="s", num_cores=NC, num_subcores=NS)


@functools.partial(
    pl.kernel,
    out_type=(
        jax.ShapeDtypeStruct((NC, CP, D), jnp.float32),
        jax.ShapeDtypeStruct((NC, CP, 16), jnp.float32),
    ),
    mesh=_mesh,
    scratch_types=dict(
        fb0=pltpu.VMEM((CH, D), jnp.float32),
        fb1=pltpu.VMEM((CH, D), jnp.float32),
        tbuf=pltpu.VMEM((RPW,), jnp.int32),
        pbuf=pltpu.VMEM((RPW,), jnp.int32),
        ibuf=pltpu.VMEM((NCHS, CH), jnp.int32),
        cbuf=pltpu.VMEM((CH, 16), jnp.float32),
        ssum=pltpu.VMEM_SHARED((CP, D), jnp.float32),
        scnt=pltpu.VMEM_SHARED((CP, 16), jnp.float32),
        fsem0=pltpu.SemaphoreType.DMA,
        fsem1=pltpu.SemaphoreType.DMA,
    ),
)
def _scatter_kernel(features, targets, pmarks, count_src, zsum, zcnt,
                    psum, pcnt, fb0, fb1, tbuf, pbuf, ibuf, cbuf,
                    ssum, scnt, fsem0, fsem1):
    c = lax.axis_index("c")
    s = lax.axis_index("s")
    wid = c * NS + s
    base = wid * RPW

    fbs, sems, descs = (fb0, fb1), (fsem0, fsem1), [None, None]
    descs[0] = pltpu.async_copy(features.at[pl.ds(base, CH)], fb0, fsem0)
    trows = pl.ds(s * TROWS, TROWS)
    zdescs = [
        pltpu.async_copy(zsum.at[trows], ssum.at[trows], fsem1),
        pltpu.async_copy(zcnt.at[trows], scnt.at[trows], fsem1),
        pltpu.async_copy(count_src, cbuf, fsem1),
    ]
    pltpu.sync_copy(targets.at[pl.ds(base, RPW)], tbuf)
    pltpu.sync_copy(pmarks.at[pl.ds(base, RPW)], pbuf)

    for k in range(RPW // 16):
        t = tbuf[pl.ds(k * 16, 16)]
        p = pbuf[pl.ds(k * 16, 16)]
        seg = jnp.where(p == 0, t, NUM_CLASSES)
        ibuf[k // (CH // 16), pl.ds((k % (CH // 16)) * 16, 16)] = seg

    for d_ in zdescs:
        d_.wait()
    plsc.subcore_barrier()

    for j in range(NCHS):
        if j + 1 < NCHS:
            nb = (j + 1) % 2
            descs[nb] = pltpu.async_copy(
                features.at[pl.ds(base + (j + 1) * CH, CH)], fbs[nb], sems[nb])
        descs[j % 2].wait()
        pltpu.sync_copy(fbs[j % 2], ssum.at[ibuf.at[j]], add=True)
        pltpu.sync_copy(cbuf, scnt.at[ibuf.at[j]], add=True)

    plsc.subcore_barrier()

    pltpu.sync_copy(ssum.at[trows], psum.at[c, trows])
    pltpu.sync_copy(scnt.at[trows], pcnt.at[c, trows])


@functools.partial(
    pl.kernel,
    out_type=(
        jax.ShapeDtypeStruct((NW, 16), jnp.float32),
        jax.ShapeDtypeStruct((NW, 16), jnp.float32),
    ),
    mesh=_mesh,
    scratch_types=dict(
        s0buf=pltpu.VMEM((TROWS, D), jnp.float32),
        c0buf=pltpu.VMEM((TROWS, 16), jnp.float32),
        c1buf=pltpu.VMEM((TROWS, 16), jnp.float32),
        stab=pltpu.VMEM_SHARED((CP, D), jnp.float32),
        tbuf=pltpu.VMEM((NCHL, CHL), jnp.int32),
        pbuf=pltpu.VMEM((RPW,), jnp.int32),
        fb0=pltpu.VMEM((CHL, D), jnp.float32),
        fb1=pltpu.VMEM((CHL, D), jnp.float32),
        gb0=pltpu.VMEM((CHL, D), jnp.float32),
        gb1=pltpu.VMEM((CHL, D), jnp.float32),
        obuf=pltpu.VMEM((16,), jnp.float32),
        obuf2=pltpu.VMEM((16,), jnp.float32),
        fsem0=pltpu.SemaphoreType.DMA,
        fsem1=pltpu.SemaphoreType.DMA,
        gsem0=pltpu.SemaphoreType.DMA,
        gsem1=pltpu.SemaphoreType.DMA,
    ),
)
def _loss_kernel(psum, pcnt, center, features, targets, pmarks, out_sq, out_np,
                 s0buf, c0buf, c1buf, stab, tbuf, pbuf,
                 fb0, fb1, gb0, gb1, obuf, obuf2,
                 fsem0, fsem1, gsem0, gsem1):
    c = lax.axis_index("c")
    s = lax.axis_index("s")
    wid = c * NS + s
    base = wid * RPW

    trows = pl.ds(s * TROWS, TROWS)
    pltpu.sync_copy(psum.at[0, trows], s0buf)
    pltpu.sync_copy(psum.at[1, trows], gb0.at[pl.ds(0, TROWS)])
    pltpu.sync_copy(pcnt.at[0, trows], c0buf)
    pltpu.sync_copy(pcnt.at[1, trows], c1buf)
    pltpu.sync_copy(center.at[trows], gb1.at[pl.ds(0, TROWS)])

    tp_descs = []
    for j in range(NCHL):
        tp_descs.append(pltpu.async_copy(
            targets.at[pl.ds(base + j * CHL, CHL)], tbuf.at[j], fsem0))
    tp_descs.append(pltpu.async_copy(
        pmarks.at[pl.ds(base, RPW)], pbuf, fsem1))

    def update_row(r, _):
        n = c0buf[r, pl.ds(0, 16)][0] + c1buf[r, pl.ds(0, 16)][0]
        has = n > 0.0
        nb = jnp.full((16,), n, jnp.float32)
        scale = (1.0 - MOMENTUM) / jnp.maximum(nb, 1.0)
        for q in range(NQ):
            cols = pl.ds(q * 16, 16)
            sm = s0buf[r, cols] + gb0[r, cols]
            cen = gb1[r, cols]
            s0buf[r, cols] = jnp.where(has, MOMENTUM * cen + scale * sm, cen)
        return 0

    lax.fori_loop(0, TROWS, update_row, 0)
    pltpu.sync_copy(s0buf, stab.at[trows])

    for d_ in tp_descs:
        d_.wait()

    plsc.subcore_barrier()

    fbs, gbs = (fb0, fb1), (gb0, gb1)
    fsems, gsems = (fsem0, fsem1), (gsem0, gsem1)
    fdescs, gdescs = [None, None], [None, None]
    fdescs[0] = pltpu.async_copy(features.at[pl.ds(base, CHL)], fb0, fsem0)
    gdescs[0] = pltpu.async_copy(stab.at[tbuf.at[0]], gb0, gsem0)

    acc = jnp.zeros((16,), jnp.float32)
    npv = jnp.zeros((16,), jnp.float32)
    for j in range(NCHL):
        if j + 1 < NCHL:
            nb = (j + 1) % 2
            fdescs[nb] = pltpu.async_copy(
                features.at[pl.ds(base + (j + 1) * CHL, CHL)], fbs[nb],
                fsems[nb])
            gdescs[nb] = pltpu.async_copy(
                stab.at[tbuf.at[j + 1]], gbs[nb], gsems[nb])
        fdescs[j % 2].wait()
        gdescs[j % 2].wait()
        fbuf, gbuf = fbs[j % 2], gbs[j % 2]

        def grp_body(g, carry):
            a, nv = carry
            mv = jnp.where(pbuf[pl.ds(j * CHL + g * 16, 16)] != 0, 1.0, 0.0)
            nv = nv + mv
            for lane in range(16):
                m = mv[lane]
                for q in range(NQ):
                    cols = pl.ds(q * 16, 16)
                    d = (fbuf[g * 16 + lane, cols]
                         - gbuf[g * 16 + lane, cols]) * m
                    a = a + d * d
            return a, nv

        acc, npv = lax.fori_loop(0, CHL // 16, grp_body, (acc, npv))

    obuf[...] = acc
    pltpu.sync_copy(obuf, out_sq.at[wid])
    obuf2[...] = npv
    pltpu.sync_copy(obuf2, out_np.at[wid])


def kernel(features, targets, pmarks, center):
    count_src = jnp.zeros((CH, 16), jnp.float32).at[:, 0].set(1.0)
    zsum = jnp.zeros((CP, D), jnp.float32)
    zcnt = jnp.zeros((CP, 16), jnp.float32)
    center_pad = jnp.zeros((CP, D), jnp.float32).at[:NUM_CLASSES].set(center)

    psum, pcnt = _scatter_kernel(features, targets, pmarks, count_src,
                                 zsum, zcnt)
    out_sq, out_np = _loss_kernel(psum, pcnt, center_pad, features, targets,
                                  pmarks)

    tot = jnp.sum(out_sq)
    n_p = jnp.sum(out_np)
    return tot / jnp.maximum(n_p * D, 1.0)


# speedup vs baseline: 1.0024x; 1.0024x over previous
import functools

import jax
import jax.numpy as jnp
from jax import lax
from jax.experimental import pallas as pl
from jax.experimental.pallas import tpu as pltpu
from jax.experimental.pallas import tpu_sc as plsc

MOMENTUM = 0.99
NUM_CLASSES = 1000
B, D = 16384, 128
CP = 1024
NC, NS = 2, 16
NW = NC * NS
RPW = B // NW
CH = 64
NCHS = RPW // CH
CHL = 128
NCHL = RPW // CHL
TROWS = CP // NS
NQ = D // 16

_mesh = plsc.VectorSubcoreMesh(
    core_axis_name="c", subcore_axis_name="s", num_cores=NC, num_subcores=NS)


@functools.partial(
    pl.kernel,
    out_type=(
        jax.ShapeDtypeStruct((NC, CP, D), jnp.float32),
        jax.ShapeDtypeStruct((NC, CP, 16), jnp.float32),
    ),
    mesh=_mesh,
    scratch_types=dict(
        fb0=pltpu.VMEM((CH, D), jnp.float32),
        fb1=pltpu.VMEM((CH, D), jnp.float32),
        tbuf=pltpu.VMEM((RPW,), jnp.int32),
        pbuf=pltpu.VMEM((RPW,), jnp.int32),
        ibuf=pltpu.VMEM((NCHS, CH), jnp.int32),
        cbuf=pltpu.VMEM((CH, 16), jnp.float32),
        ssum=pltpu.VMEM_SHARED((CP, D), jnp.float32),
        scnt=pltpu.VMEM_SHARED((CP, 16), jnp.float32),
        fsem0=pltpu.SemaphoreType.DMA,
        fsem1=pltpu.SemaphoreType.DMA,
    ),
)
def _scatter_kernel(features, targets, pmarks, count_src, zsum, zcnt,
                    psum, pcnt, fb0, fb1, tbuf, pbuf, ibuf, cbuf,
                    ssum, scnt, fsem0, fsem1):
    c = lax.axis_index("c")
    s = lax.axis_index("s")
    wid = c * NS + s
    base = wid * RPW

    fbs, sems, descs = (fb0, fb1), (fsem0, fsem1), [None, None]
    descs[0] = pltpu.async_copy(features.at[pl.ds(base, CH)], fb0, fsem0)
    trows = pl.ds(s * TROWS, TROWS)
    zdescs = [
        pltpu.async_copy(zsum.at[trows], ssum.at[trows], fsem1),
        pltpu.async_copy(zcnt.at[trows], scnt.at[trows], fsem1),
        pltpu.async_copy(count_src, cbuf, fsem1),
    ]
    pltpu.sync_copy(targets.at[pl.ds(base, RPW)], tbuf)
    pltpu.sync_copy(pmarks.at[pl.ds(base, RPW)], pbuf)

    for k in range(RPW // 16):
        t = tbuf[pl.ds(k * 16, 16)]
        p = pbuf[pl.ds(k * 16, 16)]
        seg = jnp.where(p == 0, t, NUM_CLASSES)
        ibuf[k // (CH // 16), pl.ds((k % (CH // 16)) * 16, 16)] = seg

    for d_ in zdescs:
        d_.wait()
    plsc.subcore_barrier()

    for j in range(NCHS):
        if j + 1 < NCHS:
            nb = (j + 1) % 2
            descs[nb] = pltpu.async_copy(
                features.at[pl.ds(base + (j + 1) * CH, CH)], fbs[nb], sems[nb])
        descs[j % 2].wait()
        pltpu.sync_copy(fbs[j % 2], ssum.at[ibuf.at[j]], add=True)
        pltpu.sync_copy(cbuf, scnt.at[ibuf.at[j]], add=True)

    plsc.subcore_barrier()

    pltpu.sync_copy(ssum.at[trows], psum.at[c, trows])
    pltpu.sync_copy(scnt.at[trows], pcnt.at[c, trows])


@functools.partial(
    pl.kernel,
    out_type=(
        jax.ShapeDtypeStruct((NW, 16), jnp.float32),
        jax.ShapeDtypeStruct((NW, 16), jnp.float32),
    ),
    mesh=_mesh,
    scratch_types=dict(
        s0buf=pltpu.VMEM((TROWS, D), jnp.float32),
        c0buf=pltpu.VMEM((TROWS, 16), jnp.float32),
        c1buf=pltpu.VMEM((TROWS, 16), jnp.float32),
        stab=pltpu.VMEM_SHARED((CP, D), jnp.float32),
        tbuf=pltpu.VMEM((NCHL, CHL), jnp.int32),
        pbuf=pltpu.VMEM((RPW,), jnp.int32),
        fb0=pltpu.VMEM((CHL, D), jnp.float32),
        fb1=pltpu.VMEM((CHL, D), jnp.float32),
        gb0=pltpu.VMEM((CHL, D), jnp.float32),
        gb1=pltpu.VMEM((CHL, D), jnp.float32),
        obuf=pltpu.VMEM((16,), jnp.float32),
        obuf2=pltpu.VMEM((16,), jnp.float32),
        fsem0=pltpu.SemaphoreType.DMA,
        fsem1=pltpu.SemaphoreType.DMA,
        gsem0=pltpu.SemaphoreType.DMA,
        gsem1=pltpu.SemaphoreType.DMA,
    ),
)
def _loss_kernel(psum, pcnt, center, features, targets, pmarks, out_sq, out_np,
                 s0buf, c0buf, c1buf, stab, tbuf, pbuf,
                 fb0, fb1, gb0, gb1, obuf, obuf2,
                 fsem0, fsem1, gsem0, gsem1):
    c = lax.axis_index("c")
    s = lax.axis_index("s")
    wid = c * NS + s
    base = wid * RPW

    trows = pl.ds(s * TROWS, TROWS)
    pltpu.sync_copy(psum.at[0, trows], s0buf)
    pltpu.sync_copy(psum.at[1, trows], gb0.at[pl.ds(0, TROWS)])
    pltpu.sync_copy(pcnt.at[0, trows], c0buf)
    pltpu.sync_copy(pcnt.at[1, trows], c1buf)
    pltpu.sync_copy(center.at[trows], gb1.at[pl.ds(0, TROWS)])

    tp_descs = []
    for j in range(NCHL):
        tp_descs.append(pltpu.async_copy(
            targets.at[pl.ds(base + j * CHL, CHL)], tbuf.at[j], fsem0))
    tp_descs.append(pltpu.async_copy(
        pmarks.at[pl.ds(base, RPW)], pbuf, fsem1))

    def update_row(r, _):
        n = c0buf[r, pl.ds(0, 16)][0] + c1buf[r, pl.ds(0, 16)][0]
        has = n > 0.0
        nb = jnp.full((16,), n, jnp.float32)
        scale = (1.0 - MOMENTUM) / jnp.maximum(nb, 1.0)
        for q in range(NQ):
            cols = pl.ds(q * 16, 16)
            sm = s0buf[r, cols] + gb0[r, cols]
            cen = gb1[r, cols]
            s0buf[r, cols] = jnp.where(has, MOMENTUM * cen + scale * sm, cen)
        return 0

    lax.fori_loop(0, TROWS, update_row, 0)
    pltpu.sync_copy(s0buf, stab.at[trows])

    for d_ in tp_descs:
        d_.wait()

    plsc.subcore_barrier()

    fbs, gbs = (fb0, fb1), (gb0, gb1)
    fsems, gsems = (fsem0, fsem1), (gsem0, gsem1)
    fdescs, gdescs = [None, None], [None, None]
    fdescs[0] = pltpu.async_copy(features.at[pl.ds(base, CHL)], fb0, fsem0)
    gdescs[0] = pltpu.async_copy(stab.at[tbuf.at[0]], gb0, gsem0)

    acc = jnp.zeros((16,), jnp.float32)
    npv = jnp.zeros((16,), jnp.float32)
    for j in range(NCHL):
        if j + 1 < NCHL:
            nb = (j + 1) % 2
            fdescs[nb] = pltpu.async_copy(
                features.at[pl.ds(base + (j + 1) * CHL, CHL)], fbs[nb],
                fsems[nb])
            gdescs[nb] = pltpu.async_copy(
                stab.at[tbuf.at[j + 1]], gbs[nb], gsems[nb])
        fdescs[j % 2].wait()
        gdescs[j % 2].wait()
        fbuf, gbuf = fbs[j % 2], gbs[j % 2]

        def grp_body(g, carry):
            a, nv = carry
            mv = jnp.where(pbuf[pl.ds(j * CHL + g * 16, 16)] != 0, 1.0, 0.0)
            nv = nv + mv
            for lane in range(16):
                m = mv[lane]
                for q in range(NQ):
                    cols = pl.ds(q * 16, 16)
                    d = (fbuf[g * 16 + lane, cols]
                         - gbuf[g * 16 + lane, cols]) * m
                    a = a + d * d
            return a, nv

        acc, npv = lax.fori_loop(0, CHL // 16, grp_body, (acc, npv))

    obuf[...] = acc
    pltpu.sync_copy(obuf, out_sq.at[wid])
    obuf2[...] = npv
    pltpu.sync_copy(obuf2, out_np.at[wid])


def kernel(features, targets, pmarks, center):
    count_src = jnp.zeros((CH, 16), jnp.float32).at[:, 0].set(1.0)
    zsum = jnp.zeros((CP, D), jnp.float32)
    zcnt = jnp.zeros((CP, 16), jnp.float32)
    center_pad = jnp.zeros((CP, D), jnp.float32).at[:NUM_CLASSES].set(center)

    psum, pcnt = _scatter_kernel(features, targets, pmarks, count_src,
                                 zsum, zcnt)
    out_sq, out_np = _loss_kernel(psum, pcnt, center_pad, features, targets,
                                  pmarks)

    tot = jnp.sum(out_sq)
    n_p = jnp.sum(out_np)
    return tot / jnp.maximum(n_p * D, 1.0)
